# Initial kernel scaffold; baseline (speedup 1.0000x reference)
import jax
import jax.numpy as jnp
from jax.experimental import pallas as pl


def kernel(hidden_state, Wr, Wg, W1, W2, Ws1, Ws2):
    raise NotImplementedError("write your pallas kernel here")


# speedup vs baseline: 1.2779x; 1.2779x over previous
import functools

import jax
import jax.numpy as jnp
from jax import lax
from jax.experimental import pallas as pl
from jax.experimental.pallas import tpu as pltpu

B, T, D = 2, 2048, 1024
E = 8
K = 2
DE = D // 2
N = B * T
M = N * K
BM = 256
M_PAD = M + E * BM
NT = M_PAD // BM
BT = 512
_INTERPRET = False


def _gelu(h):
    return h * 0.5 * (1.0 + lax.erf(h * 0.7071067811865475))


def _router_shared_body(x_ref, wr_ref, wg_ref, ws1_ref, ws2_ref,
                        ysh_ref, gate_ref, idx_ref):
    xb = x_ref[...]
    logits = jnp.dot(xb, wr_ref[...], preferred_element_type=jnp.float32)
    m = jnp.max(logits, axis=-1, keepdims=True)
    p = jnp.exp(logits - m)
    s = p / jnp.sum(p, axis=-1, keepdims=True)

    iota = lax.broadcasted_iota(jnp.int32, (BT, E), 1)
    top1 = jnp.max(s, axis=-1, keepdims=True)
    i1 = jnp.min(jnp.where(s == top1, iota, E), axis=-1, keepdims=True)
    s2 = jnp.where(iota == i1, -jnp.inf, s)
    top2 = jnp.max(s2, axis=-1, keepdims=True)
    i2 = jnp.min(jnp.where(s2 == top2, iota, E), axis=-1, keepdims=True)

    gate_ref[...] = jnp.concatenate([top1, top2], axis=1)
    idx_ref[...] = jnp.concatenate([i1, i2], axis=1)

    g = jnp.sum(xb * wg_ref[...], axis=-1, keepdims=True)
    h = _gelu(jnp.dot(xb, ws1_ref[...], preferred_element_type=jnp.float32))
    sh = jnp.dot(h, ws2_ref[...], preferred_element_type=jnp.float32)
    ysh_ref[...] = jax.nn.sigmoid(g) * sh


def _router_shared(x, Wr, Wg_row, Ws1, Ws2):
    nb = N // BT
    return pl.pallas_call(
        _router_shared_body,
        grid=(nb,),
        in_specs=[
            pl.BlockSpec((BT, D), lambda i: (i, 0)),
            pl.BlockSpec((D, E), lambda i: (0, 0)),
            pl.BlockSpec((1, D), lambda i: (0, 0)),
            pl.BlockSpec((D, DE), lambda i: (0, 0)),
            pl.BlockSpec((DE, D), lambda i: (0, 0)),
        ],
        out_specs=[
            pl.BlockSpec((BT, D), lambda i: (i, 0)),
            pl.BlockSpec((BT, K), lambda i: (i, 0)),
            pl.BlockSpec((BT, K), lambda i: (i, 0)),
        ],
        out_shape=[
            jax.ShapeDtypeStruct((N, D), jnp.float32),
            jax.ShapeDtypeStruct((N, K), jnp.float32),
            jax.ShapeDtypeStruct((N, K), jnp.int32),
        ],
        interpret=_INTERPRET,
    )(x, Wr, Wg_row, Ws1, Ws2)


def _gmm_body(gids_ref, xs_ref, w1_ref, w2_ref, gate_ref, ys_ref):
    xb = xs_ref[...]
    h = _gelu(jnp.dot(xb, w1_ref[0], preferred_element_type=jnp.float32))
    y = jnp.dot(h, w2_ref[0], preferred_element_type=jnp.float32)
    ys_ref[...] = y * gate_ref[...]


def _gmm(xs, W1, W2, gate_sorted, gids):
    grid_spec = pltpu.PrefetchScalarGridSpec(
        num_scalar_prefetch=1,
        grid=(NT,),
        in_specs=[
            pl.BlockSpec((BM, D), lambda i, g: (i, 0)),
            pl.BlockSpec((1, D, DE), lambda i, g: (g[i], 0, 0)),
            pl.BlockSpec((1, DE, D), lambda i, g: (g[i], 0, 0)),
            pl.BlockSpec((BM, 1), lambda i, g: (i, 0)),
        ],
        out_specs=pl.BlockSpec((BM, D), lambda i, g: (i, 0)),
    )
    return pl.pallas_call(
        _gmm_body,
        grid_spec=grid_spec,
        out_shape=jax.ShapeDtypeStruct((M_PAD, D), jnp.float32),
        interpret=_INTERPRET,
    )(gids, xs, W1, W2, gate_sorted)


def kernel(hidden_state, Wr, Wg, W1, W2, Ws1, Ws2):
    x = hidden_state.reshape(N, D)
    ysh, gate2, idx2 = _router_shared(x, Wr, Wg.reshape(1, D), Ws1, Ws2)

    e_flat = idx2.reshape(M)
    gate_flat = gate2.reshape(M)
    onehot = (e_flat[:, None] == jnp.arange(E, dtype=jnp.int32)[None, :]).astype(jnp.int32)
    counts = jnp.sum(onehot, axis=0)
    rank = jnp.take_along_axis(jnp.cumsum(onehot, axis=0) - onehot,
                               e_flat[:, None], axis=1)[:, 0]
    padded = ((counts + BM - 1) // BM) * BM
    poff = jnp.concatenate([jnp.zeros((1,), jnp.int32),
                            jnp.cumsum(padded).astype(jnp.int32)])
    pos = poff[e_flat] + rank

    tok = jnp.arange(M, dtype=jnp.int32) // K
    tok_sorted = jnp.zeros((M_PAD,), jnp.int32).at[pos].set(tok)
    gate_sorted = jnp.zeros((M_PAD,), jnp.float32).at[pos].set(gate_flat)

    xs = x[tok_sorted]
    starts = jnp.arange(NT, dtype=jnp.int32) * BM
    gids = jnp.clip(jnp.searchsorted(poff, starts, side='right') - 1, 0, E - 1)
    gids = gids.astype(jnp.int32)

    ys = _gmm(xs, W1, W2, gate_sorted.reshape(M_PAD, 1), gids)

    y = ysh + ys[pos].reshape(N, K, D).sum(axis=1)
    return y.reshape(B, T, D)


# speedup vs baseline: 1.5604x; 1.2211x over previous
import functools

import jax
import jax.numpy as jnp
from jax import lax
from jax.experimental import pallas as pl
from jax.experimental.pallas import tpu as pltpu
from jax.experimental.pallas import tpu_sc as plsc

B, T, D = 2, 2048, 1024
E = 8
K = 2
DE = D // 2
N = B * T
M = N * K
BM = 256
M_PAD = M + E * BM
NT = M_PAD // BM
BT = 512
_INTERPRET = False


def _gelu(h):
    return h * 0.5 * (1.0 + lax.erf(h * 0.7071067811865475))


def _router_shared_body(x_ref, wr_ref, wg_ref, ws1_ref, ws2_ref,
                        ysh_ref, gate_ref, idx_ref):
    xb = x_ref[...]
    logits = jnp.dot(xb, wr_ref[...], preferred_element_type=jnp.float32)
    m = jnp.max(logits, axis=-1, keepdims=True)
    p = jnp.exp(logits - m)
    s = p / jnp.sum(p, axis=-1, keepdims=True)

    iota = lax.broadcasted_iota(jnp.int32, (BT, E), 1)
    top1 = jnp.max(s, axis=-1, keepdims=True)
    i1 = jnp.min(jnp.where(s == top1, iota, E), axis=-1, keepdims=True)
    s2 = jnp.where(iota == i1, -jnp.inf, s)
    top2 = jnp.max(s2, axis=-1, keepdims=True)
    i2 = jnp.min(jnp.where(s2 == top2, iota, E), axis=-1, keepdims=True)

    gate_ref[...] = jnp.concatenate([top1, top2], axis=1)
    idx_ref[...] = jnp.concatenate([i1, i2], axis=1)

    g = jnp.sum(xb * wg_ref[...], axis=-1, keepdims=True)
    h = _gelu(jnp.dot(xb, ws1_ref[...], preferred_element_type=jnp.float32))
    sh = jnp.dot(h, ws2_ref[...], preferred_element_type=jnp.float32)
    ysh_ref[...] = jax.nn.sigmoid(g) * sh


def _router_shared(x, Wr, Wg_row, Ws1, Ws2):
    nb = N // BT
    return pl.pallas_call(
        _router_shared_body,
        grid=(nb,),
        in_specs=[
            pl.BlockSpec((BT, D), lambda i: (i, 0)),
            pl.BlockSpec((D, E), lambda i: (0, 0)),
            pl.BlockSpec((1, D), lambda i: (0, 0)),
            pl.BlockSpec((D, DE), lambda i: (0, 0)),
            pl.BlockSpec((DE, D), lambda i: (0, 0)),
        ],
        out_specs=[
            pl.BlockSpec((BT, D), lambda i: (i, 0)),
            pl.BlockSpec((BT, K), lambda i: (i, 0)),
            pl.BlockSpec((BT, K), lambda i: (i, 0)),
        ],
        out_shape=[
            jax.ShapeDtypeStruct((N, D), jnp.float32),
            jax.ShapeDtypeStruct((N, K), jnp.float32),
            jax.ShapeDtypeStruct((N, K), jnp.int32),
        ],
        interpret=_INTERPRET,
    )(x, Wr, Wg_row, Ws1, Ws2)


def _gmm_body(gids_ref, xs_ref, w1_ref, w2_ref, gate_ref, ys_ref):
    xb = xs_ref[...]
    h = _gelu(jnp.dot(xb, w1_ref[0], preferred_element_type=jnp.float32))
    y = jnp.dot(h, w2_ref[0], preferred_element_type=jnp.float32)
    ys_ref[...] = y * gate_ref[...]


def _gmm(xs, W1, W2, gate_sorted, gids):
    grid_spec = pltpu.PrefetchScalarGridSpec(
        num_scalar_prefetch=1,
        grid=(NT,),
        in_specs=[
            pl.BlockSpec((BM, D), lambda i, g: (i, 0)),
            pl.BlockSpec((1, D, DE), lambda i, g: (g[i], 0, 0)),
            pl.BlockSpec((1, DE, D), lambda i, g: (g[i], 0, 0)),
            pl.BlockSpec((BM, 1), lambda i, g: (i, 0)),
        ],
        out_specs=pl.BlockSpec((BM, D), lambda i, g: (i, 0)),
    )
    return pl.pallas_call(
        _gmm_body,
        grid_spec=grid_spec,
        out_shape=jax.ShapeDtypeStruct((M_PAD, D), jnp.float32),
        interpret=_INTERPRET,
    )(gids, xs, W1, W2, gate_sorted)


_SC_MESH = plsc.VectorSubcoreMesh(core_axis_name="c", subcore_axis_name="s")
_EPT = M // 16
_NV = _EPT // 16
_HALF = M_PAD // 2
_OPT = _HALF // 16


def _cvec(val):
    return jnp.full((16,), val, jnp.int32)


def _lane_cumsum(x, iota):
    zero = jnp.zeros((16,), jnp.int32)
    y = x
    for sh in (1, 2, 4, 8):
        idx = jnp.maximum(iota - _cvec(sh), zero)
        g = y.at[idx].get(mode='promise_in_bounds')
        y = y + jnp.where(iota >= _cvec(sh), g, zero)
    return y


def _dispatch_body(e_hbm, g_hbm, pos_hbm, tok_hbm, gate_hbm, poff_hbm,
                   ev, gv, rankv, tokv, posv, base_r, crow, allc, prow,
                   obuf_i, obuf_f, counts_sh, tok_sh, gate_sh):
    c = lax.axis_index("c")
    s = lax.axis_index("s")
    eb = s * _EPT
    iota = lax.iota(jnp.int32, 16)
    zero = jnp.zeros((16,), jnp.int32)
    one = jnp.ones((16,), jnp.int32)

    pltpu.sync_copy(e_hbm.at[pl.ds(eb, _EPT)], ev)
    for j in range(_EPT // 128):
        pltpu.sync_copy(g_hbm.at[pl.ds(eb + j * 128, 128)], gv.at[j])

    run = [jnp.zeros((16,), jnp.int32) for _ in range(E)]
    for i in range(_NV):
        v = ev[pl.ds(i * 16, 16)]
        for e in range(E):
            run[e] = run[e] + jnp.where(v == _cvec(e), one, zero)
        tok = lax.shift_right_arithmetic(
            jnp.full((16,), eb + i * 16, jnp.int32) + iota, one)
        tokv[i * 16 // 128, pl.ds((i * 16) % 128, 16)] = tok

    last = _cvec(15)
    lane_excl = []
    cv = jnp.zeros((16,), jnp.int32)
    for e in range(E):
        incl_e = _lane_cumsum(run[e], iota)
        lane_excl.append(incl_e - run[e])
        tot_e = incl_e.at[last].get(mode='promise_in_bounds')
        cv = jnp.where(iota == _cvec(e), tot_e, cv)
    crow[...] = cv
    pltpu.sync_copy(crow, counts_sh.at[s])

    run2 = [jnp.zeros((16,), jnp.int32) for _ in range(E)]
    for i in range(_NV):
        v = ev[pl.ds(i * 16, 16)]
        r = jnp.zeros((16,), jnp.int32)
        for e in range(E):
            m = v == _cvec(e)
            r = jnp.where(m, lane_excl[e] + run2[e], r)
            run2[e] = run2[e] + jnp.where(m, one, zero)
        rankv[pl.ds(i * 16, 16)] = r
    plsc.subcore_barrier()

    pltpu.sync_copy(counts_sh, allc)
    sv = jnp.full((16,), s, jnp.int32)
    tot = jnp.zeros((16,), jnp.int32)
    prefix = jnp.zeros((16,), jnp.int32)
    for t in range(16):
        row = allc[t]
        tot = tot + row
        prefix = prefix + jnp.where(_cvec(t) < sv, row, zero)
    shv = _cvec(_BM_LOG2)
    padded = lax.shift_left(
        lax.shift_right_logical(tot + _cvec(BM - 1), shv), shv)
    incl = _lane_cumsum(padded, iota)
    base_r[...] = (incl - padded) + prefix

    @pl.when(jnp.logical_and(c == 0, s == 0))
    def _():
        prow[...] = incl
        pltpu.sync_copy(prow, poff_hbm)

    for i in range(_NV):
        v = ev[pl.ds(i * 16, 16)]
        r = rankv[pl.ds(i * 16, 16)]
        b = plsc.load_gather(base_r, [v])
        posv[i * 16 // 128, pl.ds((i * 16) % 128, 16)] = b + r

    owner = jnp.logical_or(jnp.logical_and(c == 0, s < 8),
                           jnp.logical_and(c == 1, s >= 8))

    @pl.when(owner)
    def _():
        pltpu.sync_copy(posv, pos_hbm.at[pl.ds(s * (_EPT // 128), _EPT // 128)])

    for j in range(_EPT // 128):
        pltpu.sync_copy(tokv.at[j], tok_sh.at[posv.at[j]])
        pltpu.sync_copy(gv.at[j], gate_sh.at[posv.at[j]])
    plsc.subcore_barrier()

    off = c * _HALF + s * _OPT
    pltpu.sync_copy(tok_sh.at[pl.ds(off, _OPT)], obuf_i)
    pltpu.sync_copy(obuf_i, tok_hbm.at[pl.ds(off, _OPT)])
    pltpu.sync_copy(gate_sh.at[pl.ds(off, _OPT)], obuf_f)
    pltpu.sync_copy(obuf_f, gate_hbm.at[pl.ds(off, _OPT)])


_BM_LOG2 = BM.bit_length() - 1


def _dispatch_sc(e_flat, gate_flat):
    return pl.kernel(
        _dispatch_body,
        out_type=[
            jax.ShapeDtypeStruct((M // 128, 128), jnp.int32),
            jax.ShapeDtypeStruct((M_PAD,), jnp.int32),
            jax.ShapeDtypeStruct((M_PAD,), jnp.float32),
            jax.ShapeDtypeStruct((16,), jnp.int32),
        ],
        mesh=_SC_MESH,
        compiler_params=pltpu.CompilerParams(needs_layout_passes=False),
        scratch_types=[
            pltpu.VMEM((_EPT,), jnp.int32),
            pltpu.VMEM((_EPT // 128, 128), jnp.float32),
            pltpu.VMEM((_EPT,), jnp.int32),
            pltpu.VMEM((_EPT // 128, 128), jnp.int32),
            pltpu.VMEM((_EPT // 128, 128), jnp.int32),
            pltpu.VMEM((16,), jnp.int32),
            pltpu.VMEM((16,), jnp.int32),
            pltpu.VMEM((16, 16), jnp.int32),
            pltpu.VMEM((16,), jnp.int32),
            pltpu.VMEM((_OPT,), jnp.int32),
            pltpu.VMEM((_OPT,), jnp.float32),
            pltpu.VMEM_SHARED((16, 16), jnp.int32),
            pltpu.VMEM_SHARED((M_PAD,), jnp.int32),
            pltpu.VMEM_SHARED((M_PAD,), jnp.float32),
        ],
    )(e_flat, gate_flat)


def kernel(hidden_state, Wr, Wg, W1, W2, Ws1, Ws2):
    x = hidden_state.reshape(N, D)
    ysh, gate2, idx2 = _router_shared(x, Wr, Wg.reshape(1, D), Ws1, Ws2)

    e_flat = idx2.reshape(M)
    gate_flat = gate2.reshape(M)
    pos2d, tok_sorted, gate_sorted, incl = _dispatch_sc(e_flat, gate_flat)
    pos = pos2d.reshape(M)

    poff = jnp.concatenate([jnp.zeros((1,), jnp.int32), incl[:E]])
    xs = x[jnp.clip(tok_sorted, 0, N - 1)]
    starts = jnp.arange(NT, dtype=jnp.int32) * BM
    gids = jnp.clip(jnp.searchsorted(poff, starts, side='right') - 1, 0, E - 1)
    gids = gids.astype(jnp.int32)

    ys = _gmm(xs, W1, W2, gate_sorted.reshape(M_PAD, 1), gids)

    y = ysh + ys[pos].reshape(N, K, D).sum(axis=1)
    return y.reshape(B, T, D)


# speedup vs baseline: 1.8710x; 1.1990x over previous
import functools

import jax
import jax.numpy as jnp
from jax import lax
from jax.experimental import pallas as pl
from jax.experimental.pallas import tpu as pltpu
from jax.experimental.pallas import tpu_sc as plsc

B, T, D = 2, 2048, 1024
E = 8
K = 2
DE = D // 2
N = B * T
M = N * K
BM = 256
M_PAD = M + E * BM
NT = M_PAD // BM
BT = 512
_INTERPRET = False


def _gelu(h):
    return h * 0.5 * (1.0 + lax.erf(h * 0.7071067811865475))


def _router_shared_body(x_ref, wr_ref, wg_ref, ws1_ref, ws2_ref,
                        ysh_ref, gate_ref, idx_ref):
    xb = x_ref[...]
    logits = jnp.dot(xb, wr_ref[...], preferred_element_type=jnp.float32)
    m = jnp.max(logits, axis=-1, keepdims=True)
    p = jnp.exp(logits - m)
    s = p / jnp.sum(p, axis=-1, keepdims=True)

    iota = lax.broadcasted_iota(jnp.int32, (BT, E), 1)
    top1 = jnp.max(s, axis=-1, keepdims=True)
    i1 = jnp.min(jnp.where(s == top1, iota, E), axis=-1, keepdims=True)
    s2 = jnp.where(iota == i1, -jnp.inf, s)
    top2 = jnp.max(s2, axis=-1, keepdims=True)
    i2 = jnp.min(jnp.where(s2 == top2, iota, E), axis=-1, keepdims=True)

    gate_ref[...] = jnp.concatenate([top1, top2], axis=1)
    idx_ref[...] = jnp.concatenate([i1, i2], axis=1)

    g = jnp.sum(xb * wg_ref[...], axis=-1, keepdims=True)
    h = _gelu(jnp.dot(xb, ws1_ref[...], preferred_element_type=jnp.float32))
    sh = jnp.dot(h, ws2_ref[...], preferred_element_type=jnp.float32)
    ysh_ref[...] = jax.nn.sigmoid(g) * sh


def _router_shared(x, Wr, Wg_row, Ws1, Ws2):
    nb = N // BT
    return pl.pallas_call(
        _router_shared_body,
        grid=(nb,),
        in_specs=[
            pl.BlockSpec((BT, D), lambda i: (i, 0)),
            pl.BlockSpec((D, E), lambda i: (0, 0)),
            pl.BlockSpec((1, D), lambda i: (0, 0)),
            pl.BlockSpec((D, DE), lambda i: (0, 0)),
            pl.BlockSpec((DE, D), lambda i: (0, 0)),
        ],
        out_specs=[
            pl.BlockSpec((BT, D), lambda i: (i, 0)),
            pl.BlockSpec((BT, K), lambda i: (i, 0)),
            pl.BlockSpec((BT, K), lambda i: (i, 0)),
        ],
        out_shape=[
            jax.ShapeDtypeStruct((N, D), jnp.float32),
            jax.ShapeDtypeStruct((N, K), jnp.float32),
            jax.ShapeDtypeStruct((N, K), jnp.int32),
        ],
        interpret=_INTERPRET,
    )(x, Wr, Wg_row, Ws1, Ws2)


def _gmm_body(gids_ref, xs_ref, w1_ref, w2_ref, gate_ref, ys_ref):
    xb = xs_ref[...]
    h = _gelu(jnp.dot(xb, w1_ref[0], preferred_element_type=jnp.float32))
    y = jnp.dot(h, w2_ref[0], preferred_element_type=jnp.float32)
    ys_ref[...] = y * gate_ref[...]


def _gmm(xs, W1, W2, gate_sorted, gids):
    grid_spec = pltpu.PrefetchScalarGridSpec(
        num_scalar_prefetch=1,
        grid=(NT,),
        in_specs=[
            pl.BlockSpec((BM, D), lambda i, g: (i, 0)),
            pl.BlockSpec((1, D, DE), lambda i, g: (g[i], 0, 0)),
            pl.BlockSpec((1, DE, D), lambda i, g: (g[i], 0, 0)),
            pl.BlockSpec((BM, 1), lambda i, g: (i, 0)),
        ],
        out_specs=pl.BlockSpec((BM, D), lambda i, g: (i, 0)),
    )
    return pl.pallas_call(
        _gmm_body,
        grid_spec=grid_spec,
        out_shape=jax.ShapeDtypeStruct((M_PAD, D), jnp.float32),
        interpret=_INTERPRET,
    )(gids, xs, W1, W2, gate_sorted)


_SC_MESH = plsc.VectorSubcoreMesh(core_axis_name="c", subcore_axis_name="s")
_EPT = M // 16
_NV = _EPT // 16
_HALF = M_PAD // 2
_OPT = _HALF // 16


def _cvec(val):
    return jnp.full((16,), val, jnp.int32)


def _lane_cumsum(x, iota):
    zero = jnp.zeros((16,), jnp.int32)
    y = x
    for sh in (1, 2, 4, 8):
        idx = jnp.maximum(iota - _cvec(sh), zero)
        g = y.at[idx].get(mode='promise_in_bounds')
        y = y + jnp.where(iota >= _cvec(sh), g, zero)
    return y


def _dispatch_body(e_hbm, g_hbm, pos_hbm, tok_hbm, gate_hbm, poff_hbm,
                   ev, gv, rankv, tokv, posv, base_r, crow, allc, prow,
                   obuf_i, obuf_f, counts_sh, tok_sh, gate_sh):
    c = lax.axis_index("c")
    s = lax.axis_index("s")
    eb = s * _EPT
    iota = lax.iota(jnp.int32, 16)
    zero = jnp.zeros((16,), jnp.int32)
    one = jnp.ones((16,), jnp.int32)

    pltpu.sync_copy(e_hbm.at[pl.ds(eb, _EPT)], ev)
    for j in range(_EPT // 128):
        pltpu.sync_copy(g_hbm.at[pl.ds(eb + j * 128, 128)], gv.at[j])

    run = [jnp.zeros((16,), jnp.int32) for _ in range(E)]
    for i in range(_NV):
        v = ev[pl.ds(i * 16, 16)]
        for e in range(E):
            run[e] = run[e] + jnp.where(v == _cvec(e), one, zero)
        tok = lax.shift_right_arithmetic(
            jnp.full((16,), eb + i * 16, jnp.int32) + iota, one)
        tokv[i * 16 // 128, pl.ds((i * 16) % 128, 16)] = tok

    last = _cvec(15)
    lane_excl = []
    cv = jnp.zeros((16,), jnp.int32)
    for e in range(E):
        incl_e = _lane_cumsum(run[e], iota)
        lane_excl.append(incl_e - run[e])
        tot_e = incl_e.at[last].get(mode='promise_in_bounds')
        cv = jnp.where(iota == _cvec(e), tot_e, cv)
    crow[...] = cv
    pltpu.sync_copy(crow, counts_sh.at[s])

    run2 = [jnp.zeros((16,), jnp.int32) for _ in range(E)]
    for i in range(_NV):
        v = ev[pl.ds(i * 16, 16)]
        r = jnp.zeros((16,), jnp.int32)
        for e in range(E):
            m = v == _cvec(e)
            r = jnp.where(m, lane_excl[e] + run2[e], r)
            run2[e] = run2[e] + jnp.where(m, one, zero)
        rankv[pl.ds(i * 16, 16)] = r
    plsc.subcore_barrier()

    pltpu.sync_copy(counts_sh, allc)
    sv = jnp.full((16,), s, jnp.int32)
    tot = jnp.zeros((16,), jnp.int32)
    prefix = jnp.zeros((16,), jnp.int32)
    for t in range(16):
        row = allc[t]
        tot = tot + row
        prefix = prefix + jnp.where(_cvec(t) < sv, row, zero)
    shv = _cvec(_BM_LOG2)
    padded = lax.shift_left(
        lax.shift_right_logical(tot + _cvec(BM - 1), shv), shv)
    incl = _lane_cumsum(padded, iota)
    base_r[...] = (incl - padded) + prefix

    @pl.when(jnp.logical_and(c == 0, s == 0))
    def _():
        prow[...] = incl
        pltpu.sync_copy(prow, poff_hbm)

    for i in range(_NV):
        v = ev[pl.ds(i * 16, 16)]
        r = rankv[pl.ds(i * 16, 16)]
        b = plsc.load_gather(base_r, [v])
        posv[i * 16 // 128, pl.ds((i * 16) % 128, 16)] = b + r

    owner = jnp.logical_or(jnp.logical_and(c == 0, s < 8),
                           jnp.logical_and(c == 1, s >= 8))

    @pl.when(owner)
    def _():
        pltpu.sync_copy(posv, pos_hbm.at[pl.ds(s * (_EPT // 128), _EPT // 128)])

    for j in range(_EPT // 128):
        pltpu.sync_copy(tokv.at[j], tok_sh.at[posv.at[j]])
        pltpu.sync_copy(gv.at[j], gate_sh.at[posv.at[j]])
    plsc.subcore_barrier()

    off = c * _HALF + s * _OPT
    pltpu.sync_copy(tok_sh.at[pl.ds(off, _OPT)], obuf_i)
    pltpu.sync_copy(obuf_i, tok_hbm.at[pl.ds(off, _OPT)])
    pltpu.sync_copy(gate_sh.at[pl.ds(off, _OPT)], obuf_f)
    pltpu.sync_copy(obuf_f, gate_hbm.at[pl.ds(off, _OPT)])


_BM_LOG2 = BM.bit_length() - 1


def _dispatch_sc(e_flat, gate_flat):
    return pl.kernel(
        _dispatch_body,
        out_type=[
            jax.ShapeDtypeStruct((M // 128, 128), jnp.int32),
            jax.ShapeDtypeStruct((M_PAD,), jnp.int32),
            jax.ShapeDtypeStruct((M_PAD,), jnp.float32),
            jax.ShapeDtypeStruct((16,), jnp.int32),
        ],
        mesh=_SC_MESH,
        compiler_params=pltpu.CompilerParams(needs_layout_passes=False),
        scratch_types=[
            pltpu.VMEM((_EPT,), jnp.int32),
            pltpu.VMEM((_EPT // 128, 128), jnp.float32),
            pltpu.VMEM((_EPT,), jnp.int32),
            pltpu.VMEM((_EPT // 128, 128), jnp.int32),
            pltpu.VMEM((_EPT // 128, 128), jnp.int32),
            pltpu.VMEM((16,), jnp.int32),
            pltpu.VMEM((16,), jnp.int32),
            pltpu.VMEM((16, 16), jnp.int32),
            pltpu.VMEM((16,), jnp.int32),
            pltpu.VMEM((_OPT,), jnp.int32),
            pltpu.VMEM((_OPT,), jnp.float32),
            pltpu.VMEM_SHARED((16, 16), jnp.int32),
            pltpu.VMEM_SHARED((M_PAD,), jnp.int32),
            pltpu.VMEM_SHARED((M_PAD,), jnp.float32),
        ],
    )(e_flat, gate_flat)


_CT = 32
_NCH = N // 32 // _CT


def _combine_body(ys_hbm, pos_hbm, ysh_hbm, y_hbm, posv, gbuf, sbuf, sem):
    c = lax.axis_index("c")
    s = lax.axis_index("s")
    w = s * 2 + c
    tok0 = w * (_NCH * _CT)

    for j in range(_NCH):
        pltpu.sync_copy(pos_hbm.at[w * 2 + j // 2, pl.ds((j % 2) * 64, 64)],
                        posv.at[j])

    for j in range(_NCH):
        pltpu.async_copy(ys_hbm.at[posv.at[j]], gbuf, sem).wait()
        pltpu.sync_copy(ysh_hbm.at[pl.ds(tok0 + j * _CT, _CT)], sbuf)

        def body(t, carry):
            for q in range(D // 16):
                sl = pl.ds(q * 16, 16)
                sbuf[t, sl] = sbuf[t, sl] + gbuf[2 * t, sl] + gbuf[2 * t + 1, sl]
            return carry

        lax.fori_loop(0, _CT, body, 0)
        pltpu.sync_copy(sbuf, y_hbm.at[pl.ds(tok0 + j * _CT, _CT)])


def _combine_sc(ys, pos2d, ysh):
    return pl.kernel(
        _combine_body,
        out_type=jax.ShapeDtypeStruct((N, D), jnp.float32),
        mesh=_SC_MESH,
        compiler_params=pltpu.CompilerParams(needs_layout_passes=False),
        scratch_types=[
            pltpu.VMEM((_NCH, 2 * _CT), jnp.int32),
            pltpu.VMEM((2 * _CT, D), jnp.float32),
            pltpu.VMEM((_CT, D), jnp.float32),
            pltpu.SemaphoreType.DMA,
        ],
    )(ys, pos2d, ysh)


def kernel(hidden_state, Wr, Wg, W1, W2, Ws1, Ws2):
    x = hidden_state.reshape(N, D)
    ysh, gate2, idx2 = _router_shared(x, Wr, Wg.reshape(1, D), Ws1, Ws2)

    e_flat = idx2.reshape(M)
    gate_flat = gate2.reshape(M)
    pos2d, tok_sorted, gate_sorted, incl = _dispatch_sc(e_flat, gate_flat)

    poff = jnp.concatenate([jnp.zeros((1,), jnp.int32), incl[:E]])
    xs = x[jnp.clip(tok_sorted, 0, N - 1)]
    starts = jnp.arange(NT, dtype=jnp.int32) * BM
    gids = jnp.clip(jnp.searchsorted(poff, starts, side='right') - 1, 0, E - 1)
    gids = gids.astype(jnp.int32)

    ys = _gmm(xs, W1, W2, gate_sorted.reshape(M_PAD, 1), gids)

    y = _combine_sc(ys, pos2d, ysh)
    return y.reshape(B, T, D)
